# baseline (speedup 1.0000x reference)
import functools

import jax
import jax.numpy as jnp
from jax import lax
from jax.experimental import pallas as pl
from jax.experimental.pallas import tpu as pltpu
from jax.experimental.pallas import tpu_sc as plsc

N = 10000
NP = 10112
LANES = 128
TILES = 32
SUBC = 16
EB = 128
NB = 40
NBA = 64
NBB = 16
GRP = 16
DGRP = 8
EPS = 1e-5
NSLICE = NP // SUBC

_HIGH = None


def _mesh():
    return plsc.VectorSubcoreMesh(core_axis_name="c", subcore_axis_name="s",
                                  num_cores=2, num_subcores=SUBC)


def _fill(buf, value):
    def body(i, carry):
        buf[i // (LANES // 16), pl.ds((i % (LANES // 16)) * 16, 16)] = (
            jnp.full((16,), value, jnp.float32))
        return carry
    lax.fori_loop(0, EB * (LANES // 16), body, 0)


def _deg_call(cols3, zeros_h):
    @functools.partial(
        pl.kernel,
        out_type=jax.ShapeDtypeStruct((2 * NP, LANES), jnp.float32),
        mesh=_mesh(),
        scratch_types=[
            pltpu.VMEM((DGRP, EB), jnp.int32),
            pltpu.VMEM((EB, LANES), jnp.float32),
            pltpu.VMEM_SHARED((NP, LANES), jnp.float32),
        ],
    )
    def deg_kernel(cols_hbm, zeros_hbm, out_hbm, cols_v, ones_v, accum):
        cid = lax.axis_index("c")
        sid = lax.axis_index("s")
        wid = sid * 2 + cid
        base = sid * NSLICE
        _fill(ones_v, 1.0)
        pltpu.sync_copy(zeros_hbm, accum.at[pl.ds(base, NSLICE)])
        plsc.subcore_barrier()

        def group(g, carry):
            b0 = wid * NB + g * DGRP
            pltpu.sync_copy(cols_hbm.at[pl.ds(b0, DGRP)], cols_v)
            for k in range(DGRP):
                pltpu.sync_copy(ones_v, accum.at[cols_v.at[k]], add=True)
            return carry
        lax.fori_loop(0, NB // DGRP, group, 0)
        plsc.subcore_barrier()
        pltpu.sync_copy(accum.at[pl.ds(base, NSLICE)],
                        out_hbm.at[pl.ds(cid * NP + base, NSLICE)])

    return deg_kernel(cols3, zeros_h)


def _scatter_call(hp_flat, rows_hbm, cols3, zeros_h, C):
    @functools.partial(
        pl.kernel,
        out_type=jax.ShapeDtypeStruct((2 * C * NP, LANES), jnp.float32),
        mesh=_mesh(),
        scratch_types=[
            pltpu.VMEM((GRP, EB), jnp.int32),
            pltpu.VMEM((GRP, EB), jnp.int32),
            pltpu.VMEM((EB, LANES), jnp.float32),
            pltpu.VMEM((EB, LANES), jnp.float32),
            pltpu.VMEM_SHARED((NP, LANES), jnp.float32),
            pltpu.SemaphoreType.DMA,
            pltpu.SemaphoreType.DMA,
            pltpu.SemaphoreType.DMA,
            pltpu.SemaphoreType.DMA,
        ],
    )
    def scat_kernel(hp_hbm, rows_h, cols_h, zeros_hbm, out_hbm,
                    rows_v, cols_v, gbuf0, gbuf1, accum,
                    gs0, gs1, ss0, ss1):
        cid = lax.axis_index("c")
        sid = lax.axis_index("s")
        base = sid * NSLICE
        bufs = (gbuf0, gbuf1)
        gsems = (gs0, gs1)
        ssems = (ss0, ss1)
        nbatch0 = sid * (NBA + NBB) + cid * NBA
        ngroups = jnp.where(cid == 0, NBA // GRP, NBB // GRP)
        for c in range(C):
            pltpu.sync_copy(zeros_hbm, accum.at[pl.ds(base, NSLICE)])
            plsc.subcore_barrier()

            def group(g, carry):
                b0 = nbatch0 + g * GRP
                pltpu.sync_copy(rows_h.at[c, pl.ds(b0, GRP)], rows_v)
                pltpu.sync_copy(cols_h.at[pl.ds(b0, GRP)], cols_v)
                gd = [None] * GRP
                sd = [None] * GRP
                gd[0] = pltpu.async_copy(
                    hp_hbm.at[rows_v.at[0]], bufs[0], gsems[0])
                for k in range(GRP):
                    p = k % 2
                    q = (k + 1) % 2
                    if k >= 1:
                        sd[k - 1].wait()
                    if k + 1 < GRP:
                        gd[k + 1] = pltpu.async_copy(
                            hp_hbm.at[rows_v.at[k + 1]],
                            bufs[q], gsems[q])
                    gd[k].wait()
                    sd[k] = pltpu.async_copy(
                        bufs[p], accum.at[cols_v.at[k]], ssems[p],
                        add=True)
                sd[GRP - 1].wait()
                return carry
            lax.fori_loop(0, ngroups, group, 0)
            plsc.subcore_barrier()
            pltpu.sync_copy(
                accum.at[pl.ds(base, NSLICE)],
                out_hbm.at[pl.ds(cid * (C * NP) + c * NP + base, NSLICE)])
            plsc.subcore_barrier()

    return scat_kernel(hp_flat, rows_hbm, cols3, zeros_h)


def _pair_gather_call(h3, idx):
    per = 2048 // TILES

    @functools.partial(
        pl.kernel,
        out_type=jax.ShapeDtypeStruct((2048, 256), jnp.float32),
        mesh=_mesh(),
        scratch_types=[
            pltpu.VMEM((per,), jnp.int32),
            pltpu.VMEM((per, 256), jnp.float32),
            pltpu.SemaphoreType.DMA,
        ],
    )
    def gat_kernel(h_hbm, idx_hbm, out_hbm, idx_v, buf, sem):
        cid = lax.axis_index("c")
        sid = lax.axis_index("s")
        wid = sid * 2 + cid
        pltpu.sync_copy(idx_hbm.at[pl.ds(wid * per, per)], idx_v)
        pltpu.async_copy(h_hbm.at[idx_v], buf, sem).wait()
        pltpu.sync_copy(buf, out_hbm.at[pl.ds(wid * per, per)])

    return gat_kernel(h3, idx)


def _dis_body(p_ref, o_ref):
    o_ref[...] = lax.rsqrt(p_ref[0] + p_ref[1] + 1.0)


def _dis_call(deg_parts):
    return pl.pallas_call(
        _dis_body,
        out_shape=jax.ShapeDtypeStruct((NP, LANES), jnp.float32),
    )(deg_parts)


def _mm_body(x_ref, w_ref, b_ref, dis_ref, o_ref):
    h = lax.dot_general(x_ref[...], w_ref[...], (((1,), (1,)), ((), ())),
                        preferred_element_type=jnp.float32, precision=_HIGH)
    o_ref[0] = (h + b_ref[0, :1]) * dis_ref[...]


def _mm_call(act, W, br, dis, C, Fin):
    RB = 1000
    return pl.pallas_call(
        _mm_body,
        grid=(C, N // RB),
        in_specs=[
            pl.BlockSpec((RB, Fin), lambda c, i: (i, 0)),
            pl.BlockSpec((LANES, Fin), lambda c, i: (c, 0)),
            pl.BlockSpec((1, 8, LANES), lambda c, i: (c, 0, 0)),
            pl.BlockSpec((RB, LANES), lambda c, i: (i, 0)),
        ],
        out_specs=pl.BlockSpec((1, RB, LANES), lambda c, i: (c, i, 0)),
        out_shape=jax.ShapeDtypeStruct((C, N, LANES), jnp.float32),
    )(act, W, br, dis)


def _bn_body(s_ref, hp_ref, dis_ref, g_ref, be_ref, o_ref):
    z = (s_ref[0, 0, :N] + s_ref[1, 0, :N] + hp_ref[0]) * dis_ref[:N]
    mu = jnp.mean(z, axis=0, keepdims=True)
    zc = z - mu
    var = jnp.mean(zc * zc, axis=0, keepdims=True)
    y = zc * lax.rsqrt(var + EPS) * g_ref[0, :1] + be_ref[0, :1]
    o_ref[...] = jnp.where(y > 0, y, 0.1 * y)


def _bn_call(s, hp, dis, gr, ber, C):
    return pl.pallas_call(
        _bn_body,
        grid=(C,),
        in_specs=[
            pl.BlockSpec((2, 1, NP, LANES), lambda c: (0, c, 0, 0)),
            pl.BlockSpec((1, N, LANES), lambda c: (c, 0, 0)),
            pl.BlockSpec((NP, LANES), lambda c: (0, 0)),
            pl.BlockSpec((1, 8, LANES), lambda c: (c, 0, 0)),
            pl.BlockSpec((1, 8, LANES), lambda c: (c, 0, 0)),
        ],
        out_specs=pl.BlockSpec((N, LANES), lambda c: (0, c)),
        out_shape=jax.ShapeDtypeStruct((N, C * LANES), jnp.float32),
    )(s, hp, dis, gr, ber)


def _dec_body(ab_ref, p1_ref, p2_ref, o_ref):
    a = ab_ref[:1024]
    b = ab_ref[1024:]
    m1 = lax.dot_general(p1_ref[...], p2_ref[...], (((1,), (0,)), ((), ())),
                         preferred_element_type=jnp.float32, precision=_HIGH)
    m = lax.dot_general(m1, p1_ref[...], (((1,), (1,)), ((), ())),
                        preferred_element_type=jnp.float32, precision=_HIGH)
    t = lax.dot_general(a, m, (((1,), (0,)), ((), ())),
                        preferred_element_type=jnp.float32, precision=_HIGH)
    y = jnp.sum(t * b, axis=1, keepdims=True)
    o_ref[...] = jnp.broadcast_to(y, (1024, LANES))


def _dec_call(ab, P1, P2):
    return pl.pallas_call(
        _dec_body,
        out_shape=jax.ShapeDtypeStruct((1024, LANES), jnp.float32),
    )(ab, P1, P2)


def _pad8(v, C):
    return jnp.broadcast_to(v.reshape(C, 1, LANES), (C, 8, LANES))


def _layer(act, W, b, g, be, dis, rows4, cols3, zeros_h, C, Fin):
    hp = _mm_call(act, W, _pad8(b, C), dis, C, Fin)
    s = _scatter_call(hp.reshape(C * N, LANES), rows4[:C], cols3,
                      zeros_h, C)
    sr = s.reshape(2, C, NP, LANES)
    return _bn_call(sr, hp, dis, _pad8(g, C), _pad8(be, C), C)


def kernel(x, edge_index, drug_index, W1, b1, W2, b2, W3, b3,
           g1, be1, g2, be2, g3, be3, P1, P2):
    epad = TILES * NB * EB
    row = edge_index[0]
    col = edge_index[1]
    pad = epad - row.shape[0]
    rowp = jnp.concatenate([row, jnp.zeros((pad,), jnp.int32)])
    colp = jnp.concatenate([col, jnp.full((pad,), N, jnp.int32)])
    cols3 = colp.reshape(epad // EB, EB)
    rows4 = (rowp[None, :]
             + (jnp.arange(4, dtype=jnp.int32) * N)[:, None]
             ).reshape(4, epad // EB, EB)

    zeros_h = jnp.zeros((NSLICE, LANES), jnp.float32)
    deg = _deg_call(cols3, zeros_h)
    dis = _dis_call(deg.reshape(2, NP, LANES))

    a1 = _layer(x, W1, b1, g1, be1, dis, rows4, cols3, zeros_h, C=4, Fin=256)
    a2 = _layer(a1, W2, b2, g2, be2, dis, rows4, cols3, zeros_h, C=4, Fin=512)
    a3 = _layer(a2, W3, b3, g3, be3, dis, rows4, cols3, zeros_h, C=2, Fin=512)

    di = drug_index.reshape(-1, 2)
    idx = jnp.concatenate([(di[:, 0] - 1) % N, (di[:, 1] - 1) % N]
                          ).astype(jnp.int32)
    ab = _pair_gather_call(a3, idx)
    y = _dec_call(ab, P1, P2)
    return y[:, :1]


# speedup vs baseline: 1.0062x; 1.0062x over previous
import functools

import jax
import jax.numpy as jnp
from jax import lax
from jax.experimental import pallas as pl
from jax.experimental.pallas import tpu as pltpu
from jax.experimental.pallas import tpu_sc as plsc

N = 10000
NP = 10112
LANES = 128
TILES = 32
SUBC = 16
EB = 128
NB = 40
NBA = 64
NBB = 16
GRP = 16
DGRP = 8
EPS = 1e-5
NSLICE = NP // SUBC

_HIGH = None


def _mesh():
    return plsc.VectorSubcoreMesh(core_axis_name="c", subcore_axis_name="s",
                                  num_cores=2, num_subcores=SUBC)


def _fill(buf, value):
    def body(i, carry):
        buf[i // (LANES // 16), pl.ds((i % (LANES // 16)) * 16, 16)] = (
            jnp.full((16,), value, jnp.float32))
        return carry
    lax.fori_loop(0, EB * (LANES // 16), body, 0)


def _deg_call(cols3, zeros_h):
    @functools.partial(
        pl.kernel,
        out_type=jax.ShapeDtypeStruct((2 * NP, LANES), jnp.float32),
        mesh=_mesh(),
        scratch_types=[
            pltpu.VMEM((DGRP, EB), jnp.int32),
            pltpu.VMEM((EB, LANES), jnp.float32),
            pltpu.VMEM_SHARED((NP, LANES), jnp.float32),
        ],
    )
    def deg_kernel(cols_hbm, zeros_hbm, out_hbm, cols_v, ones_v, accum):
        cid = lax.axis_index("c")
        sid = lax.axis_index("s")
        wid = sid * 2 + cid
        base = sid * NSLICE
        _fill(ones_v, 1.0)
        pltpu.sync_copy(zeros_hbm, accum.at[pl.ds(base, NSLICE)])
        plsc.subcore_barrier()

        def group(g, carry):
            b0 = wid * NB + g * DGRP
            pltpu.sync_copy(cols_hbm.at[pl.ds(b0, DGRP)], cols_v)
            for k in range(DGRP):
                pltpu.sync_copy(ones_v, accum.at[cols_v.at[k]], add=True)
            return carry
        lax.fori_loop(0, NB // DGRP, group, 0)
        plsc.subcore_barrier()
        pltpu.sync_copy(accum.at[pl.ds(base, NSLICE)],
                        out_hbm.at[pl.ds(cid * NP + base, NSLICE)])

    return deg_kernel(cols3, zeros_h)


def _scatter_call(hp_flat, rows_hbm, cols3, zeros_h, C):
    @functools.partial(
        pl.kernel,
        out_type=jax.ShapeDtypeStruct((2 * C * NP, LANES), jnp.float32),
        mesh=_mesh(),
        scratch_types=[
            pltpu.VMEM((GRP, EB), jnp.int32),
            pltpu.VMEM((GRP, EB), jnp.int32),
            pltpu.VMEM((EB, LANES), jnp.float32),
            pltpu.VMEM((EB, LANES), jnp.float32),
            pltpu.VMEM_SHARED((NP, LANES), jnp.float32),
            pltpu.SemaphoreType.DMA,
            pltpu.SemaphoreType.DMA,
            pltpu.SemaphoreType.DMA,
            pltpu.SemaphoreType.DMA,
        ],
    )
    def scat_kernel(hp_hbm, rows_h, cols_h, zeros_hbm, out_hbm,
                    rows_v, cols_v, gbuf0, gbuf1, accum,
                    gs0, gs1, ss0, ss1):
        cid = lax.axis_index("c")
        sid = lax.axis_index("s")
        base = sid * NSLICE
        bufs = (gbuf0, gbuf1)
        gsems = (gs0, gs1)
        ssems = (ss0, ss1)
        nbatch0 = sid * (NBA + NBB) + cid * NBA
        ngroups = jnp.where(cid == 0, NBA // GRP, NBB // GRP)
        for c in range(C):
            pltpu.sync_copy(zeros_hbm, accum.at[pl.ds(base, NSLICE)])
            plsc.subcore_barrier()

            def group(g, carry):
                b0 = nbatch0 + g * GRP
                pltpu.sync_copy(rows_h.at[c, pl.ds(b0, GRP)], rows_v)
                pltpu.sync_copy(cols_h.at[pl.ds(b0, GRP)], cols_v)
                gd = [None] * GRP
                sd = [None] * GRP
                gd[0] = pltpu.async_copy(
                    hp_hbm.at[rows_v.at[0]], bufs[0], gsems[0])
                for k in range(GRP):
                    p = k % 2
                    q = (k + 1) % 2
                    if k >= 1:
                        sd[k - 1].wait()
                    if k + 1 < GRP:
                        gd[k + 1] = pltpu.async_copy(
                            hp_hbm.at[rows_v.at[k + 1]],
                            bufs[q], gsems[q])
                    gd[k].wait()
                    sd[k] = pltpu.async_copy(
                        bufs[p], accum.at[cols_v.at[k]], ssems[p],
                        add=True)
                sd[GRP - 1].wait()
                return carry
            lax.fori_loop(0, ngroups, group, 0)
            plsc.subcore_barrier()
            pltpu.sync_copy(
                accum.at[pl.ds(base, NSLICE)],
                out_hbm.at[pl.ds(cid * (C * NP) + c * NP + base, NSLICE)])
            plsc.subcore_barrier()

    return scat_kernel(hp_flat, rows_hbm, cols3, zeros_h)


def _pair_gather_call(h3, idx):
    per = 2048 // TILES

    @functools.partial(
        pl.kernel,
        out_type=jax.ShapeDtypeStruct((2048, 256), jnp.float32),
        mesh=_mesh(),
        scratch_types=[
            pltpu.VMEM((per,), jnp.int32),
            pltpu.VMEM((per, 256), jnp.float32),
            pltpu.SemaphoreType.DMA,
        ],
    )
    def gat_kernel(h_hbm, idx_hbm, out_hbm, idx_v, buf, sem):
        cid = lax.axis_index("c")
        sid = lax.axis_index("s")
        wid = sid * 2 + cid
        pltpu.sync_copy(idx_hbm.at[pl.ds(wid * per, per)], idx_v)
        pltpu.async_copy(h_hbm.at[idx_v], buf, sem).wait()
        pltpu.sync_copy(buf, out_hbm.at[pl.ds(wid * per, per)])

    return gat_kernel(h3, idx)


def _dis_body(p_ref, o_ref):
    o_ref[...] = lax.rsqrt(p_ref[0] + p_ref[1] + 1.0)


def _dis_call(deg_parts):
    return pl.pallas_call(
        _dis_body,
        out_shape=jax.ShapeDtypeStruct((NP, LANES), jnp.float32),
    )(deg_parts)


def _mm_body(x_ref, w_ref, b_ref, dis_ref, o_ref):
    h = lax.dot_general(x_ref[...], w_ref[...], (((1,), (1,)), ((), ())),
                        preferred_element_type=jnp.float32, precision=_HIGH)
    o_ref[0] = (h + b_ref[0, :1]) * dis_ref[...]


def _mm_call(act, W, br, dis, C, Fin):
    RB = 1000
    return pl.pallas_call(
        _mm_body,
        grid=(N // RB, C),
        in_specs=[
            pl.BlockSpec((RB, Fin), lambda i, c: (i, 0)),
            pl.BlockSpec((LANES, Fin), lambda i, c: (c, 0)),
            pl.BlockSpec((1, 8, LANES), lambda i, c: (c, 0, 0)),
            pl.BlockSpec((RB, LANES), lambda i, c: (i, 0)),
        ],
        out_specs=pl.BlockSpec((1, RB, LANES), lambda i, c: (c, i, 0)),
        out_shape=jax.ShapeDtypeStruct((C, N, LANES), jnp.float32),
    )(act, W, br, dis)


def _bn_body(s_ref, hp_ref, dis_ref, g_ref, be_ref, o_ref):
    z = (s_ref[0, 0, :N] + s_ref[1, 0, :N] + hp_ref[0]) * dis_ref[:N]
    mu = jnp.mean(z, axis=0, keepdims=True)
    zc = z - mu
    var = jnp.mean(zc * zc, axis=0, keepdims=True)
    y = zc * lax.rsqrt(var + EPS) * g_ref[0, :1] + be_ref[0, :1]
    o_ref[...] = jnp.where(y > 0, y, 0.1 * y)


def _bn_call(s, hp, dis, gr, ber, C):
    return pl.pallas_call(
        _bn_body,
        grid=(C,),
        in_specs=[
            pl.BlockSpec((2, 1, NP, LANES), lambda c: (0, c, 0, 0)),
            pl.BlockSpec((1, N, LANES), lambda c: (c, 0, 0)),
            pl.BlockSpec((NP, LANES), lambda c: (0, 0)),
            pl.BlockSpec((1, 8, LANES), lambda c: (c, 0, 0)),
            pl.BlockSpec((1, 8, LANES), lambda c: (c, 0, 0)),
        ],
        out_specs=pl.BlockSpec((N, LANES), lambda c: (0, c)),
        out_shape=jax.ShapeDtypeStruct((N, C * LANES), jnp.float32),
    )(s, hp, dis, gr, ber)


def _dec_body(ab_ref, p1_ref, p2_ref, o_ref):
    a = ab_ref[:1024]
    b = ab_ref[1024:]
    m1 = lax.dot_general(p1_ref[...], p2_ref[...], (((1,), (0,)), ((), ())),
                         preferred_element_type=jnp.float32, precision=_HIGH)
    m = lax.dot_general(m1, p1_ref[...], (((1,), (1,)), ((), ())),
                        preferred_element_type=jnp.float32, precision=_HIGH)
    t = lax.dot_general(a, m, (((1,), (0,)), ((), ())),
                        preferred_element_type=jnp.float32, precision=_HIGH)
    y = jnp.sum(t * b, axis=1, keepdims=True)
    o_ref[...] = jnp.broadcast_to(y, (1024, LANES))


def _dec_call(ab, P1, P2):
    return pl.pallas_call(
        _dec_body,
        out_shape=jax.ShapeDtypeStruct((1024, LANES), jnp.float32),
    )(ab, P1, P2)


def _pad8(v, C):
    return jnp.broadcast_to(v.reshape(C, 1, LANES), (C, 8, LANES))


def _layer(act, W, b, g, be, dis, rows4, cols3, zeros_h, C, Fin):
    hp = _mm_call(act, W, _pad8(b, C), dis, C, Fin)
    s = _scatter_call(hp.reshape(C * N, LANES), rows4[:C], cols3,
                      zeros_h, C)
    sr = s.reshape(2, C, NP, LANES)
    return _bn_call(sr, hp, dis, _pad8(g, C), _pad8(be, C), C)


def kernel(x, edge_index, drug_index, W1, b1, W2, b2, W3, b3,
           g1, be1, g2, be2, g3, be3, P1, P2):
    epad = TILES * NB * EB
    row = edge_index[0]
    col = edge_index[1]
    pad = epad - row.shape[0]
    rowp = jnp.concatenate([row, jnp.zeros((pad,), jnp.int32)])
    colp = jnp.concatenate([col, jnp.full((pad,), N, jnp.int32)])
    cols3 = colp.reshape(epad // EB, EB)
    rows4 = (rowp[None, :]
             + (jnp.arange(4, dtype=jnp.int32) * N)[:, None]
             ).reshape(4, epad // EB, EB)

    zeros_h = jnp.zeros((NSLICE, LANES), jnp.float32)
    deg = _deg_call(cols3, zeros_h)
    dis = _dis_call(deg.reshape(2, NP, LANES))

    a1 = _layer(x, W1, b1, g1, be1, dis, rows4, cols3, zeros_h, C=4, Fin=256)
    a2 = _layer(a1, W2, b2, g2, be2, dis, rows4, cols3, zeros_h, C=4, Fin=512)
    a3 = _layer(a2, W3, b3, g3, be3, dis, rows4, cols3, zeros_h, C=2, Fin=512)

    di = drug_index.reshape(-1, 2)
    idx = jnp.concatenate([(di[:, 0] - 1) % N, (di[:, 1] - 1) % N]
                          ).astype(jnp.int32)
    ab = _pair_gather_call(a3, idx)
    y = _dec_call(ab, P1, P2)
    return y[:, :1]
